# Initial kernel scaffold; baseline (speedup 1.0000x reference)
import jax
import jax.numpy as jnp
from jax.experimental import pallas as pl


def kernel(uvp, y, node_type, extend_index, ext_node_type, boundary_ghost_stencil_index):
    raise NotImplementedError("write your pallas kernel here")


# speedup vs baseline: 1.0002x; 1.0002x over previous
import jax
import jax.numpy as jnp
from jax.experimental import pallas as pl

INFLOW = 4
OUTFLOW = 5
WALL = 6
PRESS_POINT = 7


def kernel(uvp, y, node_type, extend_index, ext_node_type, boundary_ghost_stencil_index):
    mask_bc = (node_type == INFLOW) | (node_type == WALL)
    out_mask = node_type == OUTFLOW
    dummy_uv = jnp.where(mask_bc[:, None], y[:, 0:2], uvp[:, 0:2])
    dummy_p = jnp.where(out_mask, 0.0, uvp[:, 2])
    dummy = jnp.concatenate([dummy_uv, dummy_p[:, None]], axis=1)

    dummy_ext = dummy[extend_index]
    ext = uvp[extend_index]

    ghost = boundary_ghost_stencil_index[:, 0]
    n1 = boundary_ghost_stencil_index[:, 1]
    n2 = boundary_ghost_stencil_index[:, 2]
    gt = ext_node_type[ghost]
    uv_neumann = gt == OUTFLOW
    p_neumann = (gt == WALL) | (gt == INFLOW)

    new_uv = jnp.where(
        p_neumann[:, None],
        2.0 * dummy_ext[n1, 0:2] - ext[n2, 0:2],
        jnp.where(uv_neumann[:, None], ext[n2, 0:2], ext[ghost, 0:2]),
    )
    new_p = jnp.where(
        uv_neumann,
        2.0 * dummy_ext[n1, 2] - ext[n2, 2],
        jnp.where(p_neumann, ext[n2, 2], ext[ghost, 2]),
    )

    ext = ext.at[ghost, 0:2].set(new_uv)
    ext = ext.at[ghost, 2].set(new_p)

    press = ext_node_type == PRESS_POINT
    ext = ext.at[:, 2].set(jnp.where(press, 0.0, ext[:, 2]))
    return ext


# speedup vs baseline: 1.0333x; 1.0331x over previous
import functools

import jax
import jax.numpy as jnp
from jax import lax
from jax.experimental import pallas as pl
from jax.experimental.pallas import tpu as pltpu
from jax.experimental.pallas import tpu_sc as plsc

INFLOW = 4
OUTFLOW = 5
WALL = 6
PRESS_POINT = 7

NC = 2
NS = 16
NW = NC * NS
L = 16

CH = 4096


def _p1_chunk(uvp, ext_idx, ext_nt, out, idx_v, nt_v, rows_v, sem, base, n):
    pltpu.sync_copy(ext_idx.at[pl.ds(base, n)], idx_v.at[pl.ds(0, n)])
    pltpu.sync_copy(ext_nt.at[pl.ds(base, n)], nt_v.at[pl.ds(0, n)])
    descs = []
    for j in range(n // 128):
        descs.append(pltpu.async_copy(
            uvp.at[idx_v.at[pl.ds(j * 128, 128)]],
            rows_v.at[pl.ds(j * 128, 128)], sem))
    for d in descs:
        d.wait()

    col2 = jnp.full((L,), 2, jnp.int32)
    zero = jnp.zeros((L,), jnp.float32)

    @pl.loop(0, n // L)
    def _(i):
        off = i * L
        nt16 = nt_v[pl.ds(off, L)]
        press = nt16 == PRESS_POINT
        rows16 = lax.iota(jnp.int32, L) + off
        plsc.store_scatter(rows_v, [rows16, col2], zero, mask=press)

    pltpu.sync_copy(rows_v.at[pl.ds(0, n)], out.at[pl.ds(base, n)])


def _p1_body(uvp, ext_idx, ext_nt, out, idx_v, nt_v, rows_v, sem):
    c = lax.axis_index("c")
    s = lax.axis_index("s")
    w = c * NS + s
    tb = w * 37504

    for j in range(9):
        _p1_chunk(uvp, ext_idx, ext_nt, out, idx_v, nt_v, rows_v, sem,
                  tb + j * CH, CH)

    @pl.when(w != NW - 1)
    def _():
        _p1_chunk(uvp, ext_idx, ext_nt, out, idx_v, nt_v, rows_v, sem,
                  tb + 9 * CH, 640)

    @pl.when(w == NW - 1)
    def _():
        _p1_chunk(uvp, ext_idx, ext_nt, out, idx_v, nt_v, rows_v, sem,
                  tb + 9 * CH, 512)


def _p1_call(uvp, extend_index, ext_node_type):
    n_ext = extend_index.shape[0]
    mesh = plsc.VectorSubcoreMesh(
        core_axis_name="c", subcore_axis_name="s", num_cores=NC, num_subcores=NS)
    return pl.kernel(
        _p1_body,
        out_type=jax.ShapeDtypeStruct((n_ext, 3), jnp.float32),
        mesh=mesh,
        compiler_params=pltpu.CompilerParams(
            needs_layout_passes=False, use_tc_tiling_on_sc=False),
        scratch_types=[
            pltpu.VMEM((CH,), jnp.int32),
            pltpu.VMEM((CH,), jnp.int32),
            pltpu.VMEM((CH, 3), jnp.float32),
            pltpu.SemaphoreType.DMA,
        ],
    )(uvp, extend_index, ext_node_type)


def kernel(uvp, y, node_type, extend_index, ext_node_type, boundary_ghost_stencil_index):
    ext = _p1_call(uvp, extend_index, ext_node_type)

    ghost = boundary_ghost_stencil_index[:, 0]
    n1 = boundary_ghost_stencil_index[:, 1]
    n2 = boundary_ghost_stencil_index[:, 2]
    e_g = extend_index[ghost]
    e_1 = extend_index[n1]
    e_2 = extend_index[n2]
    u_g = uvp[e_g]
    u_1 = uvp[e_1]
    u_2 = uvp[e_2]
    y_1 = y[e_1]
    nt1 = node_type[e_1]
    gt = ext_node_type[ghost]

    mask_bc1 = (nt1 == INFLOW) | (nt1 == WALL)
    d1_uv = jnp.where(mask_bc1[:, None], y_1[:, 0:2], u_1[:, 0:2])
    d1_p = jnp.where(nt1 == OUTFLOW, 0.0, u_1[:, 2])

    uv_neumann = gt == OUTFLOW
    p_neumann = (gt == WALL) | (gt == INFLOW)

    new_uv = jnp.where(
        p_neumann[:, None],
        2.0 * d1_uv - u_2[:, 0:2],
        jnp.where(uv_neumann[:, None], u_2[:, 0:2], u_g[:, 0:2]),
    )
    new_p = jnp.where(
        uv_neumann,
        2.0 * d1_p - u_2[:, 2],
        jnp.where(p_neumann, u_2[:, 2], u_g[:, 2]),
    )
    new_p = jnp.where(gt == PRESS_POINT, 0.0, new_p)

    order = jnp.arange(ghost.shape[0], dtype=jnp.int32)
    last = jnp.full((ext.shape[0],), -1, jnp.int32).at[ghost].max(order)
    win = order == last[ghost]
    ghost_w = jnp.where(win, ghost, ext.shape[0])
    ext = ext.at[ghost_w, 0:2].set(new_uv, mode="drop")
    ext = ext.at[ghost_w, 2].set(new_p, mode="drop")
    return ext


# speedup vs baseline: 100.4800x; 97.2402x over previous
import functools

import jax
import jax.numpy as jnp
from jax import lax
from jax.experimental import pallas as pl
from jax.experimental.pallas import tpu as pltpu
from jax.experimental.pallas import tpu_sc as plsc

INFLOW = 4
OUTFLOW = 5
WALL = 6
PRESS_POINT = 7

NC = 2
NS = 16
NW = NC * NS
L = 16

CH = 4096
TILE_ROWS = 37504

GS = 6272
GROWS = GS // 128
G_REAL = 100000
N_EXT_C = 1200000
T_PAD = N_EXT_C + 256
HALF = 16 * TILE_ROWS
ROUNDS = 6
RCH = 1024


def _iota16():
    return lax.iota(jnp.int32, L)


def _p1_chunk(uvp, ext_idx, ext_nt, out, idx_v, nt_v, rows_v, sem, base, n):
    pltpu.sync_copy(ext_idx.at[pl.ds(base, n)], idx_v.at[pl.ds(0, n)])
    pltpu.sync_copy(ext_nt.at[pl.ds(base, n)], nt_v.at[pl.ds(0, n)])
    descs = []
    for j in range(n // 128):
        descs.append(pltpu.async_copy(
            uvp.at[idx_v.at[pl.ds(j * 128, 128)]],
            rows_v.at[pl.ds(j * 128, 128)], sem))
    for d in descs:
        d.wait()

    col2 = jnp.full((L,), 2, jnp.int32)
    zero = jnp.zeros((L,), jnp.float32)

    @pl.loop(0, n // L)
    def _(i):
        off = i * L
        press = nt_v[pl.ds(off, L)] == PRESS_POINT
        plsc.store_scatter(rows_v, [_iota16() + off, col2], zero, mask=press)

    pltpu.sync_copy(rows_v.at[pl.ds(0, n)], out.at[pl.ds(base, n)])


def _phase1(uvp, ext_idx, ext_nt, out, idx_v, nt_v, rows_v, sem, w):
    tb = w * TILE_ROWS
    for j in range(9):
        _p1_chunk(uvp, ext_idx, ext_nt, out, idx_v, nt_v, rows_v, sem,
                  tb + j * CH, CH)

    @pl.when(w != NW - 1)
    def _():
        _p1_chunk(uvp, ext_idx, ext_nt, out, idx_v, nt_v, rows_v, sem,
                  tb + 9 * CH, 640)

    @pl.when(w == NW - 1)
    def _():
        _p1_chunk(uvp, ext_idx, ext_nt, out, idx_v, nt_v, rows_v, sem,
                  tb + 9 * CH, 512)


def _dedup_body(gcol, t_out, g1_v, k1_v, gr1_v, t1_v, T_sh, sem):
    s = lax.axis_index("s")
    kbase = s * GS

    pltpu.sync_copy(gcol.at[pl.ds(kbase, GS)], g1_v)

    @pl.loop(0, GS // L)
    def _(u):
        lanes = u * L + _iota16()
        gk = kbase + lanes
        valid = gk < G_REAL
        k16 = jnp.where(valid, gk, -2)
        dum = N_EXT_C + (k16 & 255)
        gsafe = jnp.where(valid, g1_v[pl.ds(u * L, L)], dum)
        plsc.store_scatter(g1_v, [lanes], gsafe)
        plsc.store_scatter(k1_v, [lanes], k16)
        plsc.store_scatter(gr1_v, [lanes], gsafe)

    for _r in range(ROUNDS):
        @pl.loop(0, GROWS)
        def _(jb):
            o = jb * 128
            pltpu.sync_copy(k1_v.at[pl.ds(o, 128)],
                            T_sh.at[gr1_v.at[pl.ds(o, 128)]])
        plsc.subcore_barrier()

        @pl.loop(0, GROWS)
        def _(jb):
            o = jb * 128
            pltpu.async_copy(T_sh.at[gr1_v.at[pl.ds(o, 128)]],
                             t1_v.at[pl.ds(o, 128)], sem).wait()
        if _r < ROUNDS - 1:
            @pl.loop(0, GS // L)
            def _(u):
                lanes = u * L + _iota16()
                k16 = k1_v[pl.ds(u * L, L)]
                t16 = t1_v[pl.ds(u * L, L)]
                gr16 = gr1_v[pl.ds(u * L, L)]
                g16 = g1_v[pl.ds(u * L, L)]
                act = (gr16 < N_EXT_C) & (k16 > t16)
                dum = N_EXT_C + (k16 & 255)
                plsc.store_scatter(gr1_v, [lanes], jnp.where(act, g16, dum))
        plsc.subcore_barrier()

    @pl.loop(0, GROWS)
    def _(jb):
        o = jb * 128
        pltpu.async_copy(T_sh.at[g1_v.at[pl.ds(o, 128)]],
                         t1_v.at[pl.ds(o, 128)], sem).wait()

    pltpu.sync_copy(t1_v, t_out.at[pl.ds(kbase, GS)])


def _dedup_call(gcol):
    mesh = plsc.VectorSubcoreMesh(
        core_axis_name="c", subcore_axis_name="s",
        num_cores=NC, num_subcores=NS)
    i32 = jnp.int32
    return pl.kernel(
        _dedup_body,
        out_type=jax.ShapeDtypeStruct((NS * GS,), i32),
        mesh=mesh,
        compiler_params=pltpu.CompilerParams(
            needs_layout_passes=False, use_tc_tiling_on_sc=False),
        scratch_types=[
            pltpu.VMEM((GS,), i32),
            pltpu.VMEM((GS,), i32),
            pltpu.VMEM((GS,), i32),
            pltpu.VMEM((GS,), i32),
            pltpu.VMEM_SHARED((T_PAD,), i32),
            pltpu.SemaphoreType.DMA,
        ],
    )(gcol)


def _body(uvp, y, node_type, ext_idx, ext_nt, gcol, n1col, n2col, t_all, out,
          idx_v, nt_v, rows_v,
          g1_v, t1_v, n11_v, n21_v,
          cg_v, cn1_v, cn2_v,
          e1_v, e2_v, eg_v, gt_v, nt1_v,
          u1_v, u2_v, ug_v, y1_v, nr_v,
          sem):
    c = lax.axis_index("c")
    s = lax.axis_index("s")
    w = c * NS + s
    c_is1 = c == 1
    kbase = s * GS

    pltpu.sync_copy(gcol.at[pl.ds(kbase, GS)], g1_v)
    pltpu.sync_copy(n1col.at[pl.ds(kbase, GS)], n11_v)
    pltpu.sync_copy(n2col.at[pl.ds(kbase, GS)], n21_v)

    pltpu.sync_copy(t_all.at[pl.ds(kbase, GS)], t1_v)

    _phase1(uvp, ext_idx, ext_nt, out, idx_v, nt_v, rows_v, sem, w)
    plsc.subcore_barrier()

    def compact(u, m):
        lanes = u * L + _iota16()
        gk = kbase + lanes
        t16 = t1_v[pl.ds(u * L, L)]
        g16 = g1_v[pl.ds(u * L, L)]
        own = (g16 < HALF) != c_is1
        sel = (gk == t16) & own & (gk < G_REAL)
        sel_i = sel.astype(jnp.int32)
        pos = m + plsc.cumsum(sel_i) - 1
        plsc.store_scatter(cg_v, [pos], g16, mask=sel)
        plsc.store_scatter(cn1_v, [pos], n11_v[pl.ds(u * L, L)], mask=sel)
        plsc.store_scatter(cn2_v, [pos], n21_v[pl.ds(u * L, L)], mask=sel)
        return m + jnp.sum(sel_i)

    m = pl.loop(0, GS // L, init_carry=jnp.int32(0))(compact)

    @pl.when(m > 0)
    def _():
        fm = jnp.full((L,), m - 1, jnp.int32)
        lastg = plsc.load_gather(cg_v, [fm])
        lastn1 = plsc.load_gather(cn1_v, [fm])
        lastn2 = plsc.load_gather(cn2_v, [fm])
        mpad = ((m + 127) // 128) * 128

        @pl.loop(m // L, mpad // L)
        def _(u):
            pos = u * L + _iota16()
            mask = pos >= m
            plsc.store_scatter(cg_v, [pos], lastg, mask=mask)
            plsc.store_scatter(cn1_v, [pos], lastn1, mask=mask)
            plsc.store_scatter(cn2_v, [pos], lastn2, mask=mask)

        @pl.loop(0, mpad // 128)
        def _(jb):
            o = jb * 128
            cgs = cg_v.at[pl.ds(o, 128)]
            d1 = pltpu.async_copy(ext_nt.at[cgs], gt_v, sem)
            d2 = pltpu.async_copy(ext_idx.at[cgs], eg_v, sem)
            d3 = pltpu.async_copy(ext_idx.at[cn1_v.at[pl.ds(o, 128)]], e1_v, sem)
            d4 = pltpu.async_copy(ext_idx.at[cn2_v.at[pl.ds(o, 128)]], e2_v, sem)
            d1.wait(); d2.wait(); d3.wait(); d4.wait()
            d5 = pltpu.async_copy(uvp.at[eg_v], ug_v, sem)
            d6 = pltpu.async_copy(uvp.at[e1_v], u1_v, sem)
            d7 = pltpu.async_copy(uvp.at[e2_v], u2_v, sem)
            d8 = pltpu.async_copy(y.at[e1_v], y1_v, sem)
            d9 = pltpu.async_copy(node_type.at[e1_v], nt1_v, sem)
            d5.wait(); d6.wait(); d7.wait(); d8.wait(); d9.wait()

            @pl.loop(0, 128 // L)
            def _(u):
                lanes = u * L + _iota16()
                gt16 = gt_v[pl.ds(u * L, L)]
                nt116 = nt1_v[pl.ds(u * L, L)]
                uvN = gt16 == OUTFLOW
                pN = (gt16 == WALL) | (gt16 == INFLOW)
                bc1 = (nt116 == INFLOW) | (nt116 == WALL)
                out1 = nt116 == OUTFLOW

                def col(ref, cc):
                    return plsc.load_gather(
                        ref, [lanes, jnp.full((L,), cc, jnp.int32)])

                for cc in (0, 1):
                    u1c = col(u1_v, cc)
                    u2c = col(u2_v, cc)
                    ugc = col(ug_v, cc)
                    y1c = col(y1_v, cc)
                    d1c = jnp.where(bc1, y1c, u1c)
                    new = jnp.where(pN, 2.0 * d1c - u2c,
                                    jnp.where(uvN, u2c, ugc))
                    plsc.store_scatter(
                        nr_v, [lanes, jnp.full((L,), cc, jnp.int32)], new)
                u1p = col(u1_v, 2)
                u2p = col(u2_v, 2)
                ugp = col(ug_v, 2)
                d1p = jnp.where(out1, 0.0, u1p)
                newp = jnp.where(uvN, 2.0 * d1p - u2p,
                                 jnp.where(pN, u2p, ugp))
                newp = jnp.where(gt16 == PRESS_POINT, 0.0, newp)
                plsc.store_scatter(
                    nr_v, [lanes, jnp.full((L,), 2, jnp.int32)], newp)

            pltpu.sync_copy(nr_v, out.at[cgs])


def _sc_call(uvp, y, node_type, extend_index, ext_node_type,
             gcol, n1col, n2col, t_all):
    n_ext = extend_index.shape[0]
    mesh = plsc.VectorSubcoreMesh(
        core_axis_name="c", subcore_axis_name="s",
        num_cores=NC, num_subcores=NS)
    f32 = jnp.float32
    i32 = jnp.int32
    return pl.kernel(
        _body,
        out_type=jax.ShapeDtypeStruct((n_ext, 3), f32),
        mesh=mesh,
        compiler_params=pltpu.CompilerParams(
            needs_layout_passes=False, use_tc_tiling_on_sc=False),
        scratch_types=[
            pltpu.VMEM((CH,), i32),
            pltpu.VMEM((CH,), i32),
            pltpu.VMEM((CH, 3), f32),
            pltpu.VMEM((GS,), i32),
            pltpu.VMEM((GS,), i32),
            pltpu.VMEM((GS,), i32),
            pltpu.VMEM((GS,), i32),
            pltpu.VMEM((GS,), i32),
            pltpu.VMEM((GS,), i32),
            pltpu.VMEM((GS,), i32),
            pltpu.VMEM((128,), i32),
            pltpu.VMEM((128,), i32),
            pltpu.VMEM((128,), i32),
            pltpu.VMEM((128,), i32),
            pltpu.VMEM((128,), i32),
            pltpu.VMEM((128, 3), f32),
            pltpu.VMEM((128, 3), f32),
            pltpu.VMEM((128, 3), f32),
            pltpu.VMEM((128, 3), f32),
            pltpu.VMEM((128, 3), f32),
            pltpu.SemaphoreType.DMA,
        ],
    )(uvp, y, node_type, extend_index, ext_node_type, gcol, n1col, n2col,
      t_all)


def kernel(uvp, y, node_type, extend_index, ext_node_type,
           boundary_ghost_stencil_index):
    pad = NS * GS - G_REAL
    gcol = jnp.pad(boundary_ghost_stencil_index[:, 0], (0, pad))
    n1col = jnp.pad(boundary_ghost_stencil_index[:, 1], (0, pad))
    n2col = jnp.pad(boundary_ghost_stencil_index[:, 2], (0, pad))
    gcol = gcol.astype(jnp.int32)
    t_all = _dedup_call(gcol)
    return _sc_call(uvp, y, node_type, extend_index, ext_node_type,
                    gcol, n1col.astype(jnp.int32), n2col.astype(jnp.int32),
                    t_all)
